# baseline (speedup 1.0000x reference)
import functools

import jax
import jax.numpy as jnp
from jax import lax
from jax.experimental import pallas as pl
from jax.experimental.pallas import tpu as pltpu
from jax.experimental.pallas import tpu_sc as plsc

N = 10000
NP = 10112
E = 320000
D = 128

NC = 2
NS = 16
NW = NC * NS
C = 64
NCH = 160
EPW = NCH * C
EP = NW * EPW
RPT = NP // NS

BM = 1000


_MESH = plsc.VectorSubcoreMesh(core_axis_name="c", subcore_axis_name="s")


NROW = 80


def _deg_body(srci, dsti, z128, iota_h, out_s, out_d,
              sv_buf, dv_buf, hist_s, hist_d, iota_v, deg_s_sh, deg_d_sh):
    c = lax.axis_index("c")
    s = lax.axis_index("s")
    wid = c * NS + s
    base = wid * EPW
    pltpu.sync_copy(srci.at[pl.ds(base, EPW)], sv_buf)
    pltpu.sync_copy(dsti.at[pl.ds(base, EPW)], dv_buf)
    pltpu.sync_copy(z128.at[pl.ds(0, NROW)], hist_s)
    pltpu.sync_copy(z128.at[pl.ds(0, NROW)], hist_d)
    pltpu.sync_copy(iota_h, iota_v)

    def step(k, carry):
        for u in range(2):
            sv = sv_buf[pl.ds((2 * k + u) * 16, 16)]
            cnt_s, last_s = plsc.scan_count(sv)
            plsc.addupdate_scatter(
                hist_s,
                [lax.shift_right_logical(sv, 7), lax.bitwise_and(sv, 127)],
                cnt_s.astype(jnp.float32), mask=last_s)
            dv = dv_buf[pl.ds((2 * k + u) * 16, 16)]
            cnt_d, last_d = plsc.scan_count(dv)
            plsc.addupdate_scatter(
                hist_d,
                [lax.shift_right_logical(dv, 7), lax.bitwise_and(dv, 127)],
                cnt_d.astype(jnp.float32), mask=last_d)
        return carry

    lax.fori_loop(0, EPW // 32, step, 0)

    @pl.when(s == 0)
    def _():
        pltpu.sync_copy(z128.at[pl.ds(0, NROW)], deg_s_sh)
        pltpu.sync_copy(z128.at[pl.ds(0, NROW)], deg_d_sh)
    plsc.subcore_barrier()
    pltpu.sync_copy(hist_s, deg_s_sh.at[iota_v], add=True)
    pltpu.sync_copy(hist_d, deg_d_sh.at[iota_v], add=True)
    plsc.subcore_barrier()

    @pl.when(s == 0)
    def _():
        pltpu.sync_copy(deg_s_sh, out_s.at[c])
        pltpu.sync_copy(deg_d_sh, out_d.at[c])


_deg_kernel = functools.partial(
    pl.kernel,
    out_type=(jax.ShapeDtypeStruct((NC, NROW, 128), jnp.float32),
              jax.ShapeDtypeStruct((NC, NROW, 128), jnp.float32)),
    mesh=_MESH,
    compiler_params=pltpu.CompilerParams(needs_layout_passes=False),
    scratch_types=[
        pltpu.VMEM((EPW,), jnp.int32),
        pltpu.VMEM((EPW,), jnp.int32),
        pltpu.VMEM((NROW, 128), jnp.float32),
        pltpu.VMEM((NROW, 128), jnp.float32),
        pltpu.VMEM((NROW,), jnp.int32),
        pltpu.VMEM_SHARED((NROW, 128), jnp.float32),
        pltpu.VMEM_SHARED((NROW, 128), jnp.float32),
    ],
)(_deg_body)


_NBUF = 4


NPH = 5
PCH = NCH // NPH


def _scat_body(h, srcg, dstg, z128, out,
               sidx, didx, rows, agg_sh, *sems):
    gsems = sems[:_NBUF]
    ssems = sems[_NBUF:]
    c = lax.axis_index("c")
    s = lax.axis_index("s")
    wid = c * NS + s
    r0 = s * RPT
    pltpu.sync_copy(z128, agg_sh.at[pl.ds(r0, RPT)])
    plsc.subcore_barrier()

    def stage(q, p):
        return (pltpu.async_copy(srcg.at[wid, pl.ds(p * PCH, PCH)],
                                 sidx.at[q], ssems[q]),
                pltpu.async_copy(dstg.at[wid, pl.ds(p * PCH, PCH)],
                                 didx.at[q], ssems[q]))

    def stage_wait(q, p):
        pltpu.make_async_copy(srcg.at[wid, pl.ds(p * PCH, PCH)],
                              sidx.at[q], ssems[q]).wait()
        pltpu.make_async_copy(dstg.at[wid, pl.ds(p * PCH, PCH)],
                              didx.at[q], ssems[q]).wait()

    def fire(k, q, i):
        return pltpu.async_copy(h.at[sidx.at[q, i]], rows.at[k], gsems[k])

    def drain(k, q, i):
        pltpu.make_async_copy(h.at[sidx.at[q, i]], rows.at[k],
                              gsems[k]).wait()
        pltpu.sync_copy(rows.at[k], agg_sh.at[didx.at[q, i]], add=True)

    stage(0, 0)
    stage_wait(0, 0)
    for p in range(NPH):
        q = p % 2
        if p + 1 < NPH:
            stage(1 - q, p + 1)
        for k in range(_NBUF):
            fire(k, q, k)

        def step(j, carry, q=q):
            for k in range(_NBUF):
                i = _NBUF * j + k
                drain(k, q, i)
                fire(k, q, i + _NBUF)
            return carry

        lax.fori_loop(0, PCH // _NBUF - 1, step, 0)
        for k in range(_NBUF):
            drain(k, q, PCH - _NBUF + k)
        if p + 1 < NPH:
            stage_wait(1 - q, p + 1)

    plsc.subcore_barrier()
    pltpu.sync_copy(agg_sh.at[pl.ds(r0, RPT)], out.at[c, pl.ds(r0, RPT)])


_scat_kernel = functools.partial(
    pl.kernel,
    out_type=jax.ShapeDtypeStruct((NC, NP, D), jnp.float32),
    mesh=_MESH,
    scratch_types=[
        pltpu.VMEM((2, PCH, C), jnp.int32),
        pltpu.VMEM((2, PCH, C), jnp.int32),
        pltpu.VMEM((_NBUF, C, D), jnp.float32),
        pltpu.VMEM_SHARED((NP, D), jnp.float32),
    ] + [pltpu.SemaphoreType.DMA] * (_NBUF + 2),
)(_scat_body)


def _scale_body(x_ref, w_ref, ds_ref, dd_ref, h_ref, dsi_ref, ddi_ref):
    deg_s = ds_ref[0] + ds_ref[1]
    deg_d = dd_ref[0] + dd_ref[1]
    dsi = 1.0 / jnp.sqrt(jnp.maximum(deg_s, 1.0))
    ddi = 1.0 / jnp.sqrt(jnp.maximum(deg_d, 1.0))
    dsi_ref[...] = jnp.broadcast_to(dsi, dsi_ref.shape)
    ddi_ref[...] = jnp.broadcast_to(ddi, ddi_ref.shape)
    y = jnp.dot(x_ref[...], w_ref[...], preferred_element_type=jnp.float32)
    h_ref[...] = y * dsi


def _scale(x, w1, deg_s_col, deg_d_col):
    return pl.pallas_call(
        _scale_body,
        grid=(N // BM,),
        in_specs=[pl.BlockSpec((BM, D), lambda i: (i, 0)),
                  pl.BlockSpec((D, D), lambda i: (0, 0)),
                  pl.BlockSpec((NC, BM, 1), lambda i: (0, i, 0)),
                  pl.BlockSpec((NC, BM, 1), lambda i: (0, i, 0))],
        out_specs=[pl.BlockSpec((BM, D), lambda i: (i, 0)),
                   pl.BlockSpec((BM, 16), lambda i: (i, 0)),
                   pl.BlockSpec((BM, 16), lambda i: (i, 0))],
        out_shape=[jax.ShapeDtypeStruct((N, D), jnp.float32),
                   jax.ShapeDtypeStruct((N, 16), jnp.float32),
                   jax.ShapeDtypeStruct((N, 16), jnp.float32)],
    )(x, w1, deg_s_col, deg_d_col)


def _layer2_body(agg_ref, ddi_ref, b1_ref, w2_ref, dsi_ref, o_ref):
    a = agg_ref[0] + agg_ref[1]
    t = jnp.maximum(a * ddi_ref[..., :1] + b1_ref[...], 0.0)
    o_ref[...] = jnp.dot(t, w2_ref[...],
                         preferred_element_type=jnp.float32) * dsi_ref[..., :1]


def _layer2(agg1, ddi, b1_2d, w2, dsi):
    return pl.pallas_call(
        _layer2_body,
        grid=(N // BM,),
        in_specs=[pl.BlockSpec((NC, BM, D), lambda i: (0, i, 0)),
                  pl.BlockSpec((BM, 16), lambda i: (i, 0)),
                  pl.BlockSpec((1, D), lambda i: (0, 0)),
                  pl.BlockSpec((D, D), lambda i: (0, 0)),
                  pl.BlockSpec((BM, 16), lambda i: (i, 0))],
        out_specs=pl.BlockSpec((BM, D), lambda i: (i, 0)),
        out_shape=jax.ShapeDtypeStruct((N, D), jnp.float32),
    )(agg1, ddi, b1_2d, w2, dsi)


def _final_body(agg_ref, ddi_ref, b2_ref, o_ref):
    a = agg_ref[0] + agg_ref[1]
    o_ref[...] = a * ddi_ref[..., :1] + b2_ref[...]


def _final(agg2, ddi, b2_2d):
    return pl.pallas_call(
        _final_body,
        grid=(N // BM,),
        in_specs=[pl.BlockSpec((NC, BM, D), lambda i: (0, i, 0)),
                  pl.BlockSpec((BM, 16), lambda i: (i, 0)),
                  pl.BlockSpec((1, D), lambda i: (0, 0))],
        out_specs=pl.BlockSpec((BM, D), lambda i: (i, 0)),
        out_shape=jax.ShapeDtypeStruct((N, D), jnp.float32),
    )(agg2, ddi, b2_2d)


def kernel(x, edge_index, W1, b1, W2, b2):
    src = edge_index[0]
    dst = edge_index[1]
    pad = EP - E
    padi = jnp.arange(pad, dtype=jnp.int32)
    src_g = jnp.concatenate([src, padi % N])
    src_d = jnp.concatenate([src, N + (padi % 112)])
    dst_d = jnp.concatenate([dst, N + (padi % 112)])

    z128 = jnp.zeros((RPT, D), jnp.float32)
    iota_h = jnp.arange(NROW, dtype=jnp.int32)
    b1_2d = b1.reshape(1, D)
    b2_2d = b2.reshape(1, D)

    src_g3 = src_g.reshape(NW, NCH, C)
    dst_d3 = dst_d.reshape(NW, NCH, C)

    deg_s, deg_d = _deg_kernel(src_d, dst_d, z128, iota_h)
    h1s, dsi, ddi = _scale(x, W1, deg_s.reshape(NC, NROW * 128, 1),
                           deg_d.reshape(NC, NROW * 128, 1))
    agg1 = _scat_kernel(h1s, src_g3, dst_d3, z128)
    h2s = _layer2(agg1, ddi, b1_2d, W2, dsi)
    agg2 = _scat_kernel(h2s, src_g3, dst_d3, z128)
    return _final(agg2, ddi, b2_2d)


# speedup vs baseline: 1.0013x; 1.0013x over previous
import functools

import jax
import jax.numpy as jnp
from jax import lax
from jax.experimental import pallas as pl
from jax.experimental.pallas import tpu as pltpu
from jax.experimental.pallas import tpu_sc as plsc

N = 10000
NP = 10112
E = 320000
D = 128

NC = 2
NS = 16
NW = NC * NS
C = 64
NCH = 160
EPW = NCH * C
EP = NW * EPW
RPT = NP // NS

BM = 1000


_MESH = plsc.VectorSubcoreMesh(core_axis_name="c", subcore_axis_name="s")


NROW = 80


def _deg_body(srci, dsti, z128, iota_h, out_s, out_d,
              sv_buf, dv_buf, hist_s, hist_d, iota_v, deg_s_sh, deg_d_sh):
    c = lax.axis_index("c")
    s = lax.axis_index("s")
    wid = c * NS + s
    base = wid * EPW
    pltpu.sync_copy(srci.at[pl.ds(base, EPW)], sv_buf)
    pltpu.sync_copy(dsti.at[pl.ds(base, EPW)], dv_buf)
    pltpu.sync_copy(z128.at[pl.ds(0, NROW)], hist_s)
    pltpu.sync_copy(z128.at[pl.ds(0, NROW)], hist_d)
    pltpu.sync_copy(iota_h, iota_v)

    def step(k, carry):
        sv = sv_buf[pl.ds(k * 16, 16)]
        cnt_s, last_s = plsc.scan_count(sv)
        plsc.addupdate_scatter(
            hist_s,
            [lax.shift_right_logical(sv, 7), lax.bitwise_and(sv, 127)],
            cnt_s.astype(jnp.float32), mask=last_s)
        dv = dv_buf[pl.ds(k * 16, 16)]
        cnt_d, last_d = plsc.scan_count(dv)
        plsc.addupdate_scatter(
            hist_d,
            [lax.shift_right_logical(dv, 7), lax.bitwise_and(dv, 127)],
            cnt_d.astype(jnp.float32), mask=last_d)
        return carry

    lax.fori_loop(0, EPW // 16, step, 0)

    @pl.when(s == 0)
    def _():
        pltpu.sync_copy(z128.at[pl.ds(0, NROW)], deg_s_sh)
        pltpu.sync_copy(z128.at[pl.ds(0, NROW)], deg_d_sh)
    plsc.subcore_barrier()
    pltpu.sync_copy(hist_s, deg_s_sh.at[iota_v], add=True)
    pltpu.sync_copy(hist_d, deg_d_sh.at[iota_v], add=True)
    plsc.subcore_barrier()

    @pl.when(s == 0)
    def _():
        pltpu.sync_copy(deg_s_sh, out_s.at[c])
        pltpu.sync_copy(deg_d_sh, out_d.at[c])


_deg_kernel = functools.partial(
    pl.kernel,
    out_type=(jax.ShapeDtypeStruct((NC, NROW, 128), jnp.float32),
              jax.ShapeDtypeStruct((NC, NROW, 128), jnp.float32)),
    mesh=_MESH,
    compiler_params=pltpu.CompilerParams(needs_layout_passes=False),
    scratch_types=[
        pltpu.VMEM((EPW,), jnp.int32),
        pltpu.VMEM((EPW,), jnp.int32),
        pltpu.VMEM((NROW, 128), jnp.float32),
        pltpu.VMEM((NROW, 128), jnp.float32),
        pltpu.VMEM((NROW,), jnp.int32),
        pltpu.VMEM_SHARED((NROW, 128), jnp.float32),
        pltpu.VMEM_SHARED((NROW, 128), jnp.float32),
    ],
)(_deg_body)


_NBUF = 4


NPH = 5
PCH = NCH // NPH


def _scat_body(h, srcg, dstg, z128, out,
               sidx, didx, rows, agg_sh, *sems):
    gsems = sems[:_NBUF]
    ssems = sems[_NBUF:]
    c = lax.axis_index("c")
    s = lax.axis_index("s")
    wid = c * NS + s
    r0 = s * RPT
    pltpu.sync_copy(z128, agg_sh.at[pl.ds(r0, RPT)])
    plsc.subcore_barrier()

    def stage(q, p):
        return (pltpu.async_copy(srcg.at[wid, pl.ds(p * PCH, PCH)],
                                 sidx.at[q], ssems[q]),
                pltpu.async_copy(dstg.at[wid, pl.ds(p * PCH, PCH)],
                                 didx.at[q], ssems[q]))

    def stage_wait(q, p):
        pltpu.make_async_copy(srcg.at[wid, pl.ds(p * PCH, PCH)],
                              sidx.at[q], ssems[q]).wait()
        pltpu.make_async_copy(dstg.at[wid, pl.ds(p * PCH, PCH)],
                              didx.at[q], ssems[q]).wait()

    def fire(k, q, i):
        return pltpu.async_copy(h.at[sidx.at[q, i]], rows.at[k], gsems[k])

    def drain(k, q, i):
        pltpu.make_async_copy(h.at[sidx.at[q, i]], rows.at[k],
                              gsems[k]).wait()
        pltpu.sync_copy(rows.at[k], agg_sh.at[didx.at[q, i]], add=True)

    stage(0, 0)
    stage_wait(0, 0)
    for p in range(NPH):
        q = p % 2
        if p + 1 < NPH:
            stage(1 - q, p + 1)
        for k in range(_NBUF):
            fire(k, q, k)

        def step(j, carry, q=q):
            for k in range(_NBUF):
                i = _NBUF * j + k
                drain(k, q, i)
                fire(k, q, i + _NBUF)
            return carry

        lax.fori_loop(0, PCH // _NBUF - 1, step, 0)
        for k in range(_NBUF):
            drain(k, q, PCH - _NBUF + k)
        if p + 1 < NPH:
            stage_wait(1 - q, p + 1)

    plsc.subcore_barrier()
    pltpu.sync_copy(agg_sh.at[pl.ds(r0, RPT)], out.at[c, pl.ds(r0, RPT)])


_scat_kernel = functools.partial(
    pl.kernel,
    out_type=jax.ShapeDtypeStruct((NC, NP, D), jnp.float32),
    mesh=_MESH,
    scratch_types=[
        pltpu.VMEM((2, PCH, C), jnp.int32),
        pltpu.VMEM((2, PCH, C), jnp.int32),
        pltpu.VMEM((_NBUF, C, D), jnp.float32),
        pltpu.VMEM_SHARED((NP, D), jnp.float32),
    ] + [pltpu.SemaphoreType.DMA] * (_NBUF + 2),
)(_scat_body)


def _scale_body(x_ref, w_ref, ds_ref, dd_ref, h_ref, dsi_ref, ddi_ref):
    deg_s = ds_ref[0] + ds_ref[1]
    deg_d = dd_ref[0] + dd_ref[1]
    dsi = 1.0 / jnp.sqrt(jnp.maximum(deg_s, 1.0))
    ddi = 1.0 / jnp.sqrt(jnp.maximum(deg_d, 1.0))
    dsi_ref[...] = jnp.broadcast_to(dsi, dsi_ref.shape)
    ddi_ref[...] = jnp.broadcast_to(ddi, ddi_ref.shape)
    y = jnp.dot(x_ref[...], w_ref[...], preferred_element_type=jnp.float32)
    h_ref[...] = y * dsi


def _scale(x, w1, deg_s_col, deg_d_col):
    return pl.pallas_call(
        _scale_body,
        grid=(N // BM,),
        in_specs=[pl.BlockSpec((BM, D), lambda i: (i, 0)),
                  pl.BlockSpec((D, D), lambda i: (0, 0)),
                  pl.BlockSpec((NC, BM, 1), lambda i: (0, i, 0)),
                  pl.BlockSpec((NC, BM, 1), lambda i: (0, i, 0))],
        out_specs=[pl.BlockSpec((BM, D), lambda i: (i, 0)),
                   pl.BlockSpec((BM, 16), lambda i: (i, 0)),
                   pl.BlockSpec((BM, 16), lambda i: (i, 0))],
        out_shape=[jax.ShapeDtypeStruct((N, D), jnp.float32),
                   jax.ShapeDtypeStruct((N, 16), jnp.float32),
                   jax.ShapeDtypeStruct((N, 16), jnp.float32)],
    )(x, w1, deg_s_col, deg_d_col)


def _layer2_body(agg_ref, ddi_ref, b1_ref, w2_ref, dsi_ref, o_ref):
    a = agg_ref[0] + agg_ref[1]
    t = jnp.maximum(a * ddi_ref[..., :1] + b1_ref[...], 0.0)
    o_ref[...] = jnp.dot(t, w2_ref[...],
                         preferred_element_type=jnp.float32) * dsi_ref[..., :1]


def _layer2(agg1, ddi, b1_2d, w2, dsi):
    return pl.pallas_call(
        _layer2_body,
        grid=(N // BM,),
        in_specs=[pl.BlockSpec((NC, BM, D), lambda i: (0, i, 0)),
                  pl.BlockSpec((BM, 16), lambda i: (i, 0)),
                  pl.BlockSpec((1, D), lambda i: (0, 0)),
                  pl.BlockSpec((D, D), lambda i: (0, 0)),
                  pl.BlockSpec((BM, 16), lambda i: (i, 0))],
        out_specs=pl.BlockSpec((BM, D), lambda i: (i, 0)),
        out_shape=jax.ShapeDtypeStruct((N, D), jnp.float32),
    )(agg1, ddi, b1_2d, w2, dsi)


def _final_body(agg_ref, ddi_ref, b2_ref, o_ref):
    a = agg_ref[0] + agg_ref[1]
    o_ref[...] = a * ddi_ref[..., :1] + b2_ref[...]


def _final(agg2, ddi, b2_2d):
    return pl.pallas_call(
        _final_body,
        grid=(N // BM,),
        in_specs=[pl.BlockSpec((NC, BM, D), lambda i: (0, i, 0)),
                  pl.BlockSpec((BM, 16), lambda i: (i, 0)),
                  pl.BlockSpec((1, D), lambda i: (0, 0))],
        out_specs=pl.BlockSpec((BM, D), lambda i: (i, 0)),
        out_shape=jax.ShapeDtypeStruct((N, D), jnp.float32),
    )(agg2, ddi, b2_2d)


def kernel(x, edge_index, W1, b1, W2, b2):
    src = edge_index[0]
    dst = edge_index[1]
    pad = EP - E
    padi = jnp.arange(pad, dtype=jnp.int32)
    src_g = jnp.concatenate([src, padi % N])
    src_d = jnp.concatenate([src, N + (padi % 112)])
    dst_d = jnp.concatenate([dst, N + (padi % 112)])

    z128 = jnp.zeros((RPT, D), jnp.float32)
    iota_h = jnp.arange(NROW, dtype=jnp.int32)
    b1_2d = b1.reshape(1, D)
    b2_2d = b2.reshape(1, D)

    src_g3 = src_g.reshape(NW, NCH, C)
    dst_d3 = dst_d.reshape(NW, NCH, C)

    deg_s, deg_d = _deg_kernel(src_d, dst_d, z128, iota_h)
    h1s, dsi, ddi = _scale(x, W1, deg_s.reshape(NC, NROW * 128, 1),
                           deg_d.reshape(NC, NROW * 128, 1))
    agg1 = _scat_kernel(h1s, src_g3, dst_d3, z128)
    h2s = _layer2(agg1, ddi, b1_2d, W2, dsi)
    agg2 = _scat_kernel(h2s, src_g3, dst_d3, z128)
    return _final(agg2, ddi, b2_2d)
